# Initial kernel scaffold; baseline (speedup 1.0000x reference)
import functools

import jax
import jax.numpy as jnp
from jax import lax
from jax.experimental import pallas as pl
from jax.experimental.pallas import tpu as pltpu
from jax.experimental.pallas import tpu_sc as plsc

N = 320000
D = 128
S = 10000

NC = 2
NS = 16
NW = NC * NS
ROWS_PER_W = N // NW
K = 80
NCHUNK = ROWS_PER_W // K
S_PER_TILE = S // NS


def _sc_body(x_hbm, ids_hbm, zeros_hbm, out_hbm, acc, xbuf, ibuf):
    c = lax.axis_index("c")
    s = lax.axis_index("s")
    w = c * NS + s
    row0 = w * ROWS_PER_W
    seg0 = s * S_PER_TILE

    pltpu.sync_copy(zeros_hbm.at[pl.ds(seg0, S_PER_TILE)],
                    acc.at[pl.ds(seg0, S_PER_TILE)])
    plsc.subcore_barrier()

    def body(i, carry):
        base = row0 + i * K
        pltpu.sync_copy(x_hbm.at[pl.ds(base, K)], xbuf)
        pltpu.sync_copy(ids_hbm.at[pl.ds(base, K)], ibuf)
        pltpu.sync_copy(xbuf, acc.at[ibuf], add=True)
        return carry

    lax.fori_loop(0, NCHUNK, body, 0)
    plsc.subcore_barrier()

    pltpu.sync_copy(acc.at[pl.ds(seg0, S_PER_TILE)],
                    out_hbm.at[c, pl.ds(seg0, S_PER_TILE)])


_sc_pool = functools.partial(
    pl.kernel,
    mesh=plsc.VectorSubcoreMesh(core_axis_name="c", subcore_axis_name="s",
                                num_cores=NC, num_subcores=NS),
    out_type=jax.ShapeDtypeStruct((NC, S, D), jnp.float32),
    scratch_types=[
        pltpu.VMEM_SHARED((S, D), jnp.float32),
        pltpu.VMEM((K, D), jnp.float32),
        pltpu.VMEM((K,), jnp.int32),
    ],
)(_sc_body)


def _combine_body(a_ref, b_ref, o_ref):
    o_ref[...] = a_ref[0] + b_ref[0]


_BLK = 1000


def kernel(x, edge_list):
    ids = edge_list.astype(jnp.int32)
    zeros = jnp.zeros((S, D), jnp.float32)
    partials = _sc_pool(x, ids, zeros)
    out = pl.pallas_call(
        _combine_body,
        grid=(S // _BLK,),
        in_specs=[
            pl.BlockSpec((1, _BLK, D), lambda i: (0, i, 0)),
            pl.BlockSpec((1, _BLK, D), lambda i: (1, i, 0)),
        ],
        out_specs=pl.BlockSpec((_BLK, D), lambda i: (i, 0)),
        out_shape=jax.ShapeDtypeStruct((S, D), jnp.float32),
    )(partials, partials)
    return out


# speedup vs baseline: 3.7254x; 3.7254x over previous
import functools

import jax
import jax.numpy as jnp
from jax import lax
from jax.experimental import pallas as pl
from jax.experimental.pallas import tpu as pltpu
from jax.experimental.pallas import tpu_sc as plsc

N = 320000
D = 128
S = 10000
SP = 10112

NC = 2
NS = 16
NW = NC * NS
ROWS_PER_W = N // NW
K = 80
NCHUNK = ROWS_PER_W // K
S_PER_TILE = SP // NS


def _sc_body(x_hbm, ids_hbm, zeros_hbm, out_hbm, acc, xbuf, ibuf):
    c = lax.axis_index("c")
    s = lax.axis_index("s")
    w = c * NS + s
    row0 = w * ROWS_PER_W
    seg0 = s * S_PER_TILE

    pltpu.sync_copy(zeros_hbm.at[pl.ds(seg0, S_PER_TILE)],
                    acc.at[pl.ds(seg0, S_PER_TILE)])
    plsc.subcore_barrier()

    def body(i, carry):
        base = row0 + i * K
        pltpu.sync_copy(x_hbm.at[pl.ds(base, K)], xbuf)
        pltpu.sync_copy(ids_hbm.at[pl.ds(base, K)], ibuf)
        pltpu.sync_copy(xbuf, acc.at[ibuf], add=True)
        return carry

    lax.fori_loop(0, NCHUNK, body, 0)
    plsc.subcore_barrier()

    pltpu.sync_copy(acc.at[pl.ds(seg0, S_PER_TILE)],
                    out_hbm.at[c, pl.ds(seg0, S_PER_TILE)])


_sc_pool = functools.partial(
    pl.kernel,
    mesh=plsc.VectorSubcoreMesh(core_axis_name="c", subcore_axis_name="s"),
    out_type=jax.ShapeDtypeStruct((NC, SP, D), jnp.float32),
    scratch_types=[
        pltpu.VMEM_SHARED((SP, D), jnp.float32),
        pltpu.VMEM((K, D), jnp.float32),
        pltpu.VMEM((K,), jnp.int32),
    ],
)(_sc_body)


def _combine_body(a_ref, b_ref, o_ref):
    o_ref[...] = a_ref[0] + b_ref[0]


_BLK = 1000


def kernel(x, edge_list):
    ids = edge_list.astype(jnp.int32)
    zeros = jnp.zeros((SP, D), jnp.float32)
    partials = _sc_pool(x, ids, zeros)
    out = pl.pallas_call(
        _combine_body,
        grid=(S // _BLK,),
        in_specs=[
            pl.BlockSpec((1, _BLK, D), lambda i: (0, i, 0)),
            pl.BlockSpec((1, _BLK, D), lambda i: (1, i, 0)),
        ],
        out_specs=pl.BlockSpec((_BLK, D), lambda i: (i, 0)),
        out_shape=jax.ShapeDtypeStruct((S, D), jnp.float32),
    )(partials, partials)
    return out


# speedup vs baseline: 5.8925x; 1.5817x over previous
import functools

import jax
import jax.numpy as jnp
from jax import lax
from jax.experimental import pallas as pl
from jax.experimental.pallas import tpu as pltpu
from jax.experimental.pallas import tpu_sc as plsc

N = 320000
D = 128
S = 10000
SP = 10112

NC = 2
NS = 16
NW = NC * NS
ROWS_PER_W = N // NW
K = 80
NCHUNK = ROWS_PER_W // K
S_PER_TILE = SP // NS


def _sc_body(x_hbm, ids_hbm, zeros_hbm, out_hbm, acc,
             xbuf0, xbuf1, ibuf0, ibuf1, semx0, semx1, semi0, semi1):
    c = lax.axis_index("c")
    s = lax.axis_index("s")
    w = c * NS + s
    row0 = w * ROWS_PER_W
    seg0 = s * S_PER_TILE

    xbufs = (xbuf0, xbuf1)
    ibufs = (ibuf0, ibuf1)
    semxs = (semx0, semx1)
    semis = (semi0, semi1)

    def start(i, b):
        base = row0 + i * K
        pltpu.async_copy(x_hbm.at[pl.ds(base, K)], xbufs[b], semxs[b])
        pltpu.async_copy(ids_hbm.at[pl.ds(base, K)], ibufs[b], semis[b])

    def wait(b):
        pltpu.make_async_copy(x_hbm.at[pl.ds(0, K)], xbufs[b], semxs[b]).wait()
        pltpu.make_async_copy(ids_hbm.at[pl.ds(0, K)], ibufs[b], semis[b]).wait()

    start(0, 0)
    pltpu.sync_copy(zeros_hbm.at[pl.ds(seg0, S_PER_TILE)],
                    acc.at[pl.ds(seg0, S_PER_TILE)])
    plsc.subcore_barrier()

    def body(g, carry):
        for b in range(2):
            cur = 2 * g + b
            wait(b)
            start(cur + 1, 1 - b)
            pltpu.sync_copy(xbufs[b], acc.at[ibufs[b]], add=True)
        return carry

    lax.fori_loop(0, (NCHUNK - 1) // 2, body, 0)
    wait(0)
    pltpu.sync_copy(xbufs[0], acc.at[ibufs[0]], add=True)
    plsc.subcore_barrier()

    pltpu.sync_copy(acc.at[pl.ds(seg0, S_PER_TILE)],
                    out_hbm.at[c, pl.ds(seg0, S_PER_TILE)])


_sc_pool = functools.partial(
    pl.kernel,
    mesh=plsc.VectorSubcoreMesh(core_axis_name="c", subcore_axis_name="s"),
    out_type=jax.ShapeDtypeStruct((NC, SP, D), jnp.float32),
    scratch_types=[
        pltpu.VMEM_SHARED((SP, D), jnp.float32),
        pltpu.VMEM((K, D), jnp.float32),
        pltpu.VMEM((K, D), jnp.float32),
        pltpu.VMEM((K,), jnp.int32),
        pltpu.VMEM((K,), jnp.int32),
        pltpu.SemaphoreType.DMA,
        pltpu.SemaphoreType.DMA,
        pltpu.SemaphoreType.DMA,
        pltpu.SemaphoreType.DMA,
    ],
)(_sc_body)


def _combine_body(a_ref, b_ref, o_ref):
    o_ref[...] = a_ref[0] + b_ref[0]


_BLK = 1000


def kernel(x, edge_list):
    ids = edge_list.astype(jnp.int32)
    zeros = jnp.zeros((SP, D), jnp.float32)
    partials = _sc_pool(x, ids, zeros)
    out = pl.pallas_call(
        _combine_body,
        grid=(S // _BLK,),
        in_specs=[
            pl.BlockSpec((1, _BLK, D), lambda i: (0, i, 0)),
            pl.BlockSpec((1, _BLK, D), lambda i: (1, i, 0)),
        ],
        out_specs=pl.BlockSpec((_BLK, D), lambda i: (i, 0)),
        out_shape=jax.ShapeDtypeStruct((S, D), jnp.float32),
    )(partials, partials)
    return out


# speedup vs baseline: 7.6862x; 1.3044x over previous
import functools

import jax
import jax.numpy as jnp
from jax import lax
from jax.experimental import pallas as pl
from jax.experimental.pallas import tpu as pltpu
from jax.experimental.pallas import tpu_sc as plsc

N = 320000
D = 128
S = 10000
SP = 10112

NC = 2
NS = 16
NW = NC * NS
ROWS_PER_W = N // NW
K = 80
NCHUNK = ROWS_PER_W // K
S_PER_TILE = SP // NS


NB = 4
NGRP = (NCHUNK - 1) // NB


def _sc_body(x_hbm, ids_hbm, zeros_hbm, out_hbm, acc,
             xbufs, ibufs, semls, semss):
    c = lax.axis_index("c")
    s = lax.axis_index("s")
    w = c * NS + s
    row0 = w * ROWS_PER_W
    seg0 = s * S_PER_TILE

    def start_load(i, b):
        base = row0 + i * K
        pltpu.async_copy(x_hbm.at[pl.ds(base, K)], xbufs[b], semls[b])
        pltpu.async_copy(ids_hbm.at[pl.ds(base, K)], ibufs[b], semls[b])

    def wait_load(b):
        pltpu.make_async_copy(x_hbm.at[pl.ds(0, K)], xbufs[b], semls[b]).wait()
        pltpu.make_async_copy(ids_hbm.at[pl.ds(0, K)], ibufs[b],
                              semls[b]).wait()

    def start_scatter(b):
        pltpu.async_copy(xbufs[b], acc.at[ibufs[b]], semss[b], add=True)

    def wait_scatter(b):
        pltpu.make_async_copy(xbufs[b], acc.at[ibufs[b]], semss[b]).wait()

    for b in range(NB):
        start_load(b, b)
    pltpu.sync_copy(zeros_hbm.at[pl.ds(seg0, S_PER_TILE)],
                    acc.at[pl.ds(seg0, S_PER_TILE)])
    plsc.subcore_barrier()

    def body(g, carry):
        for b in range(NB):
            wait_load(b)
            start_scatter(b)
        for b in range(NB):
            wait_scatter(b)
            start_load((g + 1) * NB + b, b)
        return carry

    lax.fori_loop(0, NGRP - 1, body, 0)
    for b in range(NB):
        wait_load(b)
        start_scatter(b)
    wait_scatter(0)
    start_load(NCHUNK - 1, 0)
    wait_load(0)
    start_scatter(0)
    for b in range(1, NB):
        wait_scatter(b)
    wait_scatter(0)
    plsc.subcore_barrier()

    pltpu.sync_copy(acc.at[pl.ds(seg0, S_PER_TILE)],
                    out_hbm.at[c, pl.ds(seg0, S_PER_TILE)])


_sc_pool = functools.partial(
    pl.kernel,
    mesh=plsc.VectorSubcoreMesh(core_axis_name="c", subcore_axis_name="s"),
    out_type=jax.ShapeDtypeStruct((NC, SP, D), jnp.float32),
    scratch_types=[
        pltpu.VMEM_SHARED((SP, D), jnp.float32),
        [pltpu.VMEM((K, D), jnp.float32)] * NB,
        [pltpu.VMEM((K,), jnp.int32)] * NB,
        [pltpu.SemaphoreType.DMA] * NB,
        [pltpu.SemaphoreType.DMA] * NB,
    ],
)(_sc_body)


def _combine_body(a_ref, b_ref, o_ref):
    o_ref[...] = a_ref[0] + b_ref[0]


_BLK = 1000


def kernel(x, edge_list):
    ids = edge_list.astype(jnp.int32)
    zeros = jnp.zeros((SP, D), jnp.float32)
    partials = _sc_pool(x, ids, zeros)
    out = pl.pallas_call(
        _combine_body,
        grid=(S // _BLK,),
        in_specs=[
            pl.BlockSpec((1, _BLK, D), lambda i: (0, i, 0)),
            pl.BlockSpec((1, _BLK, D), lambda i: (1, i, 0)),
        ],
        out_specs=pl.BlockSpec((_BLK, D), lambda i: (i, 0)),
        out_shape=jax.ShapeDtypeStruct((S, D), jnp.float32),
    )(partials, partials)
    return out


# speedup vs baseline: 8.2177x; 1.0691x over previous
import functools

import jax
import jax.numpy as jnp
from jax import lax
from jax.experimental import pallas as pl
from jax.experimental.pallas import tpu as pltpu
from jax.experimental.pallas import tpu_sc as plsc

N = 320000
D = 128
S = 10000
SP = 10112

NC = 2
NS = 16
HC = D // NC
ROWS_PER_T = N // NS
K = 80
NCHUNK = ROWS_PER_T // K
S_PER_TILE = SP // NS
S_LAST = S - (NS - 1) * S_PER_TILE

NB = 5
NGRP = NCHUNK // NB


def _sc_body(x_hbm, ids_hbm, zeros_hbm, out_hbm, acc,
             xbufs, ibufs, semls, semss):
    c = lax.axis_index("c")
    s = lax.axis_index("s")
    row0 = s * ROWS_PER_T
    seg0 = s * S_PER_TILE
    col0 = c * HC

    def start_load(i, b):
        base = row0 + i * K
        pltpu.async_copy(x_hbm.at[pl.ds(base, K), pl.ds(col0, HC)],
                         xbufs[b], semls[b])
        pltpu.async_copy(ids_hbm.at[pl.ds(base, K)], ibufs[b], semls[b])

    def wait_load(b):
        pltpu.make_async_copy(x_hbm.at[pl.ds(0, K), pl.ds(0, HC)],
                              xbufs[b], semls[b]).wait()
        pltpu.make_async_copy(ids_hbm.at[pl.ds(0, K)], ibufs[b],
                              semls[b]).wait()

    def start_scatter(b):
        pltpu.async_copy(xbufs[b], acc.at[ibufs[b]], semss[b], add=True)

    def wait_scatter(b):
        pltpu.make_async_copy(xbufs[b], acc.at[ibufs[b]], semss[b]).wait()

    for b in range(NB):
        start_load(b, b)
    pltpu.sync_copy(zeros_hbm.at[pl.ds(seg0, S_PER_TILE)],
                    acc.at[pl.ds(seg0, S_PER_TILE)])
    plsc.subcore_barrier()

    def body(g, carry):
        for b in range(NB):
            wait_load(b)
            start_scatter(b)
        for b in range(NB):
            wait_scatter(b)
            start_load((g + 1) * NB + b, b)
        return carry

    lax.fori_loop(0, NGRP - 1, body, 0)
    for b in range(NB):
        wait_load(b)
        start_scatter(b)
    for b in range(NB):
        wait_scatter(b)
    plsc.subcore_barrier()

    @pl.when(s < NS - 1)
    def _():
        pltpu.sync_copy(acc.at[pl.ds(seg0, S_PER_TILE)],
                        out_hbm.at[pl.ds(seg0, S_PER_TILE), pl.ds(col0, HC)])

    @pl.when(s == NS - 1)
    def _():
        pltpu.sync_copy(acc.at[pl.ds(seg0, S_LAST)],
                        out_hbm.at[pl.ds(seg0, S_LAST), pl.ds(col0, HC)])


_sc_pool = functools.partial(
    pl.kernel,
    mesh=plsc.VectorSubcoreMesh(core_axis_name="c", subcore_axis_name="s"),
    out_type=jax.ShapeDtypeStruct((S, D), jnp.float32),
    compiler_params=pltpu.CompilerParams(use_tc_tiling_on_sc=False),
    scratch_types=[
        pltpu.VMEM_SHARED((SP, HC), jnp.float32),
        [pltpu.VMEM((K, HC), jnp.float32)] * NB,
        [pltpu.VMEM((K,), jnp.int32)] * NB,
        [pltpu.SemaphoreType.DMA] * NB,
        [pltpu.SemaphoreType.DMA] * NB,
    ],
)(_sc_body)


def kernel(x, edge_list):
    ids = edge_list.astype(jnp.int32)
    zeros = jnp.zeros((SP, HC), jnp.float32)
    return _sc_pool(x, ids, zeros)


# speedup vs baseline: 8.6462x; 1.0521x over previous
import functools

import jax
import jax.numpy as jnp
from jax import lax
from jax.experimental import pallas as pl
from jax.experimental.pallas import tpu as pltpu
from jax.experimental.pallas import tpu_sc as plsc

N = 320000
D = 128
S = 10000
SP = 10112

NC = 2
NS = 16
HC = D // NC
ROWS_PER_T = N // NS
K = 80
NCHUNK = ROWS_PER_T // K
S_PER_TILE = SP // NS
S_LAST = S - (NS - 1) * S_PER_TILE

NB = 10
NGRP = NCHUNK // NB


def _sc_body(x_hbm, ids_hbm, zeros_hbm, out_hbm, acc,
             xbufs, ibufs, semls, semss):
    c = lax.axis_index("c")
    s = lax.axis_index("s")
    row0 = s * ROWS_PER_T
    seg0 = s * S_PER_TILE
    col0 = c * HC

    def start_load(i, b):
        base = row0 + i * K
        pltpu.async_copy(x_hbm.at[pl.ds(base, K), pl.ds(col0, HC)],
                         xbufs[b], semls[b])
        pltpu.async_copy(ids_hbm.at[pl.ds(base, K)], ibufs[b], semls[b])

    def wait_load(b):
        pltpu.make_async_copy(x_hbm.at[pl.ds(0, K), pl.ds(0, HC)],
                              xbufs[b], semls[b]).wait()
        pltpu.make_async_copy(ids_hbm.at[pl.ds(0, K)], ibufs[b],
                              semls[b]).wait()

    def start_scatter(b):
        pltpu.async_copy(xbufs[b], acc.at[ibufs[b]], semss[b], add=True)

    def wait_scatter(b):
        pltpu.make_async_copy(xbufs[b], acc.at[ibufs[b]], semss[b]).wait()

    for b in range(NB):
        start_load(b, b)
    pltpu.sync_copy(zeros_hbm.at[pl.ds(seg0, S_PER_TILE)],
                    acc.at[pl.ds(seg0, S_PER_TILE)])
    plsc.subcore_barrier()

    def body(g, carry):
        for b in range(NB):
            wait_load(b)
            start_scatter(b)
        for b in range(NB):
            wait_scatter(b)
            start_load((g + 1) * NB + b, b)
        return carry

    lax.fori_loop(0, NGRP - 1, body, 0)
    for b in range(NB):
        wait_load(b)
        start_scatter(b)
    for b in range(NB):
        wait_scatter(b)
    plsc.subcore_barrier()

    @pl.when(s < NS - 1)
    def _():
        pltpu.sync_copy(acc.at[pl.ds(seg0, S_PER_TILE)],
                        out_hbm.at[pl.ds(seg0, S_PER_TILE), pl.ds(col0, HC)])

    @pl.when(s == NS - 1)
    def _():
        pltpu.sync_copy(acc.at[pl.ds(seg0, S_LAST)],
                        out_hbm.at[pl.ds(seg0, S_LAST), pl.ds(col0, HC)])


_sc_pool = functools.partial(
    pl.kernel,
    mesh=plsc.VectorSubcoreMesh(core_axis_name="c", subcore_axis_name="s"),
    out_type=jax.ShapeDtypeStruct((S, D), jnp.float32),
    compiler_params=pltpu.CompilerParams(use_tc_tiling_on_sc=False),
    scratch_types=[
        pltpu.VMEM_SHARED((SP, HC), jnp.float32),
        [pltpu.VMEM((K, HC), jnp.float32)] * NB,
        [pltpu.VMEM((K,), jnp.int32)] * NB,
        [pltpu.SemaphoreType.DMA] * NB,
        [pltpu.SemaphoreType.DMA] * NB,
    ],
)(_sc_body)


def kernel(x, edge_list):
    ids = edge_list.astype(jnp.int32)
    zeros = jnp.zeros((SP, HC), jnp.float32)
    return _sc_pool(x, ids, zeros)
